# baseline (speedup 1.0000x reference)
import functools

import jax
import jax.numpy as jnp
from jax import lax
from jax.experimental import pallas as pl
from jax.experimental.pallas import tpu as pltpu
from jax.experimental.pallas import tpu_sc as plsc

_LANES = 16
_NW = 32


def _sc_gather_cur_prev(flat, pos, batch, h, chunk_base, chunk_rows):
    bpw = chunk_rows // _NW
    mesh = plsc.VectorSubcoreMesh(core_axis_name="c", subcore_axis_name="s")

    @functools.partial(
        pl.kernel,
        mesh=mesh,
        out_type=(
            jax.ShapeDtypeStruct((chunk_rows, h), jnp.float32),
            jax.ShapeDtypeStruct((chunk_rows, h), jnp.float32),
        ),
        scratch_types=[
            pltpu.VMEM((bpw,), jnp.int32),
            pltpu.VMEM((2 * bpw,), jnp.int32),
            pltpu.VMEM((2 * bpw, h), jnp.float32),
            pltpu.SemaphoreType.DMA,
        ],
    )
    def gather_k(flat_hbm, pos_hbm, cur_out, prev_out, pos_v, idx_v, rows_v, sem):
        wid = lax.axis_index("s") * 2 + lax.axis_index("c")
        base = wid * bpw
        pltpu.sync_copy(pos_hbm.at[pl.ds(chunk_base + base, bpw)], pos_v)
        for j in range(bpw // _LANES):
            p = pos_v[pl.ds(j * _LANES, _LANES)]
            lane = lax.iota(jnp.int32, _LANES) + (chunk_base + base + j * _LANES)
            cur_idx = p * batch + lane
            idx_v[pl.ds(j * _LANES, _LANES)] = cur_idx
            idx_v[pl.ds(bpw + j * _LANES, _LANES)] = cur_idx - batch
        pltpu.async_copy(flat_hbm.at[idx_v], rows_v, sem).wait()
        pltpu.sync_copy(rows_v.at[pl.ds(0, bpw)], cur_out.at[pl.ds(base, bpw)])
        pltpu.sync_copy(rows_v.at[pl.ds(bpw, bpw)], prev_out.at[pl.ds(base, bpw)])

    return gather_k(flat, pos)


def _tc_compose(cur, prev, x, W, b2, opdir, chunk_blk, n_blk, bb, alias_outs):
    batch, h = x.shape

    def body(cur_ref, prev_ref, x_ref, w_ref, b_ref, od_ref, *rest):
        hid_ref, red_ref = rest[-2], rest[-1]
        cur_v = cur_ref[...]
        prev_v = prev_ref[...]
        w = w_ref[...]
        bvec = b_ref[...]
        dn = (((1,), (1,)), ((), ()))
        cc_l = jnp.concatenate([cur_v, prev_v], axis=1)
        cc_r = jnp.concatenate([prev_v, cur_v], axis=1)
        left = jnp.tanh(
            lax.dot_general(cc_l, w, dn, preferred_element_type=jnp.float32) + bvec)
        right = jnp.tanh(
            lax.dot_general(cc_r, w, dn, preferred_element_type=jnp.float32) + bvec)
        opv = od_ref[:, 0:1]
        drv = od_ref[:, 1:2]
        is_left = (opv == -1) & (drv == 0)
        is_right = (opv == -1) & (drv == 1)
        zero = jnp.zeros_like(left)
        red_ref[...] = jnp.where(is_left, left, jnp.where(is_right, right, zero))
        comp = jnp.where(drv == 0, left, right)
        hid_ref[...] = jnp.where(opv == 1, x_ref[...], jnp.where(opv == 0, cur_v, comp))

    in_specs = [
        pl.BlockSpec((bb, h), lambda i: (i, 0)),
        pl.BlockSpec((bb, h), lambda i: (i, 0)),
        pl.BlockSpec((bb, h), lambda i: (chunk_blk + i, 0)),
        pl.BlockSpec((h, 2 * h), lambda i: (0, 0)),
        pl.BlockSpec((1, h), lambda i: (0, 0)),
        pl.BlockSpec((bb, 2), lambda i: (chunk_blk + i, 0)),
    ]
    operands = [cur, prev, x, W, b2, opdir]
    kwargs = {}
    if alias_outs is not None:
        in_specs += [pl.BlockSpec(memory_space=pltpu.HBM),
                     pl.BlockSpec(memory_space=pltpu.HBM)]
        operands += [alias_outs[0], alias_outs[1]]
        kwargs["input_output_aliases"] = {6: 0, 7: 1}

    return pl.pallas_call(
        body,
        grid=(n_blk,),
        in_specs=in_specs,
        out_specs=[
            pl.BlockSpec((bb, h), lambda i: (chunk_blk + i, 0)),
            pl.BlockSpec((bb, h), lambda i: (chunk_blk + i, 0)),
        ],
        out_shape=[
            jax.ShapeDtypeStruct((batch, h), jnp.float32),
            jax.ShapeDtypeStruct((batch, h), jnp.float32),
        ],
        **kwargs,
    )(*operands)


def kernel(hidden_stack, x, pos, op, dir_, W, b):
    seq2, batch, h = hidden_stack.shape
    flat = hidden_stack.reshape(seq2 * batch, h)
    pos32 = pos.astype(jnp.int32)
    opdir = jnp.stack([op.astype(jnp.int32), dir_.astype(jnp.int32)], axis=1)
    b2 = b.reshape(1, h)
    half = batch // 2
    bb = 256
    n_blk = half // bb

    cur_a, prev_a = _sc_gather_cur_prev(flat, pos32, batch, h, 0, half)
    cur_b, prev_b = _sc_gather_cur_prev(flat, pos32, batch, h, half, half)
    hid_a, red_a = _tc_compose(cur_a, prev_a, x, W, b2, opdir, 0, n_blk, bb, None)
    hid, red = _tc_compose(cur_b, prev_b, x, W, b2, opdir, n_blk, n_blk, bb,
                           (hid_a, red_a))
    return hid, red


# speedup vs baseline: 1.1174x; 1.1174x over previous
import functools

import jax
import jax.numpy as jnp
from jax import lax
from jax.experimental import pallas as pl
from jax.experimental.pallas import tpu as pltpu
from jax.experimental.pallas import tpu_sc as plsc

_LANES = 16
_NW = 32


def _sc_gather_cur_prev(flat, pos, batch, h):
    bpw = batch // _NW
    mesh = plsc.VectorSubcoreMesh(core_axis_name="c", subcore_axis_name="s")

    @functools.partial(
        pl.kernel,
        mesh=mesh,
        out_type=(
            jax.ShapeDtypeStruct((batch, h), jnp.float32),
            jax.ShapeDtypeStruct((batch, h), jnp.float32),
        ),
        scratch_types=[
            pltpu.VMEM((bpw,), jnp.int32),
            pltpu.VMEM((bpw,), jnp.int32),
            pltpu.VMEM((bpw,), jnp.int32),
            pltpu.VMEM((bpw, h), jnp.float32),
            pltpu.VMEM((bpw, h), jnp.float32),
            pltpu.SemaphoreType.DMA,
            pltpu.SemaphoreType.DMA,
            pltpu.SemaphoreType.DMA,
            pltpu.SemaphoreType.DMA,
        ],
    )
    def gather_k(flat_hbm, pos_hbm, cur_out, prev_out,
                 pos_v, idxc_v, idxp_v, rows_c, rows_p, s0, s1, s2, s3):
        wid = lax.axis_index("s") * 2 + lax.axis_index("c")
        base = wid * bpw
        pltpu.sync_copy(pos_hbm.at[pl.ds(base, bpw)], pos_v)
        for j in range(bpw // _LANES):
            p = pos_v[pl.ds(j * _LANES, _LANES)]
            lane = lax.iota(jnp.int32, _LANES) + (base + j * _LANES)
            cur_idx = p * batch + lane
            idxc_v[pl.ds(j * _LANES, _LANES)] = cur_idx
            idxp_v[pl.ds(j * _LANES, _LANES)] = cur_idx - batch
        g_c = pltpu.async_copy(flat_hbm.at[idxc_v], rows_c, s0)
        g_p = pltpu.async_copy(flat_hbm.at[idxp_v], rows_p, s1)
        g_c.wait()
        w_c = pltpu.async_copy(rows_c, cur_out.at[pl.ds(base, bpw)], s2)
        g_p.wait()
        w_p = pltpu.async_copy(rows_p, prev_out.at[pl.ds(base, bpw)], s3)
        w_c.wait()
        w_p.wait()

    return gather_k(flat, pos)


def _tc_compose(cur, prev, x, W, b2, opdir):
    batch, h = x.shape
    bb = 512
    dn = (((1,), (1,)), ((), ()))

    def body(cur_ref, prev_ref, x_ref, w_ref, b_ref, od_ref, hid_ref, red_ref):
        cur_v = cur_ref[...]
        prev_v = prev_ref[...]
        w = w_ref[...]
        bvec = b_ref[...]
        cc_l = jnp.concatenate([cur_v, prev_v], axis=1)
        cc_r = jnp.concatenate([prev_v, cur_v], axis=1)
        left = jnp.tanh(
            lax.dot_general(cc_l, w, dn, preferred_element_type=jnp.float32) + bvec)
        right = jnp.tanh(
            lax.dot_general(cc_r, w, dn, preferred_element_type=jnp.float32) + bvec)
        opv = od_ref[:, 0:1]
        drv = od_ref[:, 1:2]
        is_left = (opv == -1) & (drv == 0)
        is_right = (opv == -1) & (drv == 1)
        zero = jnp.zeros_like(left)
        red_ref[...] = jnp.where(is_left, left, jnp.where(is_right, right, zero))
        comp = jnp.where(drv == 0, left, right)
        hid_ref[...] = jnp.where(opv == 1, x_ref[...], jnp.where(opv == 0, cur_v, comp))

    return pl.pallas_call(
        body,
        grid=(batch // bb,),
        in_specs=[
            pl.BlockSpec((bb, h), lambda i: (i, 0)),
            pl.BlockSpec((bb, h), lambda i: (i, 0)),
            pl.BlockSpec((bb, h), lambda i: (i, 0)),
            pl.BlockSpec((h, 2 * h), lambda i: (0, 0)),
            pl.BlockSpec((1, h), lambda i: (0, 0)),
            pl.BlockSpec((bb, 2), lambda i: (i, 0)),
        ],
        out_specs=[
            pl.BlockSpec((bb, h), lambda i: (i, 0)),
            pl.BlockSpec((bb, h), lambda i: (i, 0)),
        ],
        out_shape=[
            jax.ShapeDtypeStruct((batch, h), jnp.float32),
            jax.ShapeDtypeStruct((batch, h), jnp.float32),
        ],
    )(cur, prev, x, W, b2, opdir)


def kernel(hidden_stack, x, pos, op, dir_, W, b):
    seq2, batch, h = hidden_stack.shape
    flat = hidden_stack.reshape(seq2 * batch, h)
    pos32 = pos.astype(jnp.int32)
    cur, prev = _sc_gather_cur_prev(flat, pos32, batch, h)
    opdir = jnp.stack([op.astype(jnp.int32), dir_.astype(jnp.int32)], axis=1)
    hid, red = _tc_compose(cur, prev, x, W, b.reshape(1, h), opdir)
    return hid, red
